# Initial kernel scaffold; baseline (speedup 1.0000x reference)
import jax
import jax.numpy as jnp
from jax.experimental import pallas as pl


def kernel(structure, sequence, t, alpha):
    raise NotImplementedError("write your pallas kernel here")


# speedup vs baseline: 1.8348x; 1.8348x over previous
import functools

import numpy as np
import jax
import jax.numpy as jnp
from jax.experimental import pallas as pl
from jax.experimental.pallas import tpu as pltpu

_T = 500
_STRUC_VOCAB = 516
_SEQ_VOCAB = 33
_STRUC_MASK = 2
_SEQ_MASK = 32
_EPS = 1e-6
_TINY = float(np.finfo(np.float32).tiny)

_KEY_STRUC = (1832780943, 270669613)
_KEY_SEQ = (64467757, 2916123636)

_ROT_A = (13, 15, 26, 6)
_ROT_B = (17, 29, 16, 24)


def _rotl(x, d):
    return (x << jnp.uint32(d)) | (x >> jnp.uint32(32 - d))


def _threefry_bits(k0, k1, lo):
    ks0 = jnp.uint32(k0)
    ks1 = jnp.uint32(k1)
    ks2 = jnp.uint32(k0 ^ k1 ^ 0x1BD11BDA)
    x0 = jnp.zeros_like(lo) + ks0
    x1 = lo + ks1
    schedule = ((_ROT_A, ks1, ks2, 1), (_ROT_B, ks2, ks0, 2),
                (_ROT_A, ks0, ks1, 3), (_ROT_B, ks1, ks2, 4),
                (_ROT_A, ks2, ks0, 5))
    for rots, ka, kb, c in schedule:
        for r in rots:
            x0 = x0 + x1
            x1 = _rotl(x1, r)
            x1 = x0 ^ x1
        x0 = x0 + ka
        x1 = x1 + kb + jnp.uint32(c)
    return x0 ^ x1


def _gumbel_from_mbits(mbits):
    tiny = jnp.float32(_TINY)
    scale = jnp.float32(1.0) - tiny
    fb = jax.lax.bitcast_convert_type(
        mbits | jnp.int32(0x3F800000), jnp.float32) - jnp.float32(1.0)
    u = jnp.maximum(tiny, fb * scale + tiny)
    return -jnp.log(-jnp.log(u))


def _sample_body(tok_ref, a_ref, out_ref, *, n_vocab, n_pad, mask_tok, width,
                 key):
    q = pl.program_id(0)
    x0 = tok_ref[0]
    a = a_ref[0]

    n_ids = jax.lax.broadcasted_iota(jnp.int32, (n_pad, width), 0)
    lane = jax.lax.broadcasted_iota(jnp.uint32, (n_pad, width), 1)
    row = jnp.uint32(q * width) + lane
    flat = row * jnp.uint32(n_vocab) + n_ids.astype(jnp.uint32)

    bits = _threefry_bits(key[0], key[1], flat)
    mbits = (bits >> jnp.uint32(9)).astype(jnp.int32)

    is_x0 = n_ids == x0
    is_mask = n_ids == mask_tok
    excl = is_x0 | is_mask | (n_ids >= n_vocab)

    masked = jnp.where(excl, jnp.int32(-1), mbits)
    m_oth = jnp.max(masked, axis=0, keepdims=True)
    eligible = (~excl) & (masked == m_oth)
    n_oth = jnp.min(jnp.where(eligible, n_ids, jnp.int32(1 << 30)),
                    axis=0, keepdims=True)
    m_x0 = jnp.max(jnp.where(is_x0, mbits, jnp.int32(-1)), axis=0,
                   keepdims=True)
    m_mask = mbits[mask_tok:mask_tok + 1, :]

    g_x0 = _gumbel_from_mbits(m_x0)
    g_mask = _gumbel_from_mbits(m_mask)
    g_oth = _gumbel_from_mbits(m_oth)

    eps = jnp.float32(_EPS)
    one = jnp.float32(1.0)
    x0_is_mask = x0 == mask_tok
    p_mask = jnp.where(x0_is_mask, a + (one - a), one - a)
    c_x0 = jnp.where(x0_is_mask, jnp.float32(-jnp.inf),
                     g_x0 + jnp.log(a + eps))
    c_mask = g_mask + jnp.log(p_mask + eps)
    c_oth = g_oth + jnp.log(a * jnp.float32(0.0) + eps)

    best_v, best_i = c_x0, x0
    mask_i = jnp.full_like(best_i, mask_tok)
    take = (c_mask > best_v) | ((c_mask == best_v) & (mask_i < best_i))
    best_v = jnp.where(take, c_mask, best_v)
    best_i = jnp.where(take, mask_i, best_i)
    take = (c_oth > best_v) | ((c_oth == best_v) & (n_oth < best_i))
    best_i = jnp.where(take, n_oth, best_i)

    out_ref[0] = best_i


def _q_sample(tokens, a_rows, n_vocab, n_pad, mask_tok, key, width=512,
              interpret=False):
    b, l = tokens.shape
    rows = b * l
    grid = rows // width
    tok3 = tokens.reshape(grid, 1, width).astype(jnp.int32)
    a3 = a_rows.reshape(grid, 1, width).astype(jnp.float32)
    body = functools.partial(_sample_body, n_vocab=n_vocab, n_pad=n_pad,
                             mask_tok=mask_tok, width=width, key=key)
    out = pl.pallas_call(
        body,
        grid=(grid,),
        in_specs=[
            pl.BlockSpec((1, 1, width), lambda q: (q, 0, 0)),
            pl.BlockSpec((1, 1, width), lambda q: (q, 0, 0)),
        ],
        out_specs=pl.BlockSpec((1, 1, width), lambda q: (q, 0, 0)),
        out_shape=jax.ShapeDtypeStruct((grid, 1, width), jnp.int32),
        compiler_params=pltpu.CompilerParams(
            dimension_semantics=("parallel",)),
        interpret=interpret,
    )(tok3, a3)
    return out.reshape(b, l)


def kernel(structure, sequence, t, alpha):
    b, l = structure.shape
    a = alpha[t]
    a_rows = jnp.broadcast_to(a[:, None], (b, l))
    noised_structure = _q_sample(structure, a_rows, _STRUC_VOCAB, 520,
                                 _STRUC_MASK, _KEY_STRUC)
    noised_seq = _q_sample(sequence, a_rows, _SEQ_VOCAB, 40,
                           _SEQ_MASK, _KEY_SEQ)
    return (noised_structure, noised_seq, t)


# speedup vs baseline: 2.8301x; 1.5425x over previous
import functools

import numpy as np
import jax
import jax.numpy as jnp
from jax.experimental import pallas as pl
from jax.experimental.pallas import tpu as pltpu

_STRUC_VOCAB = 516
_SEQ_VOCAB = 33
_STRUC_MASK = 2
_SEQ_MASK = 32
_EPS = 1e-6
_TINY = float(np.finfo(np.float32).tiny)
_BIG = 1 << 30

_KEY_STRUC = (1832780943, 270669613)
_KEY_SEQ = (64467757, 2916123636)

_ROT_A = (13, 15, 26, 6)
_ROT_B = (17, 29, 16, 24)


def _rotl(x, d):
    return (x << jnp.uint32(d)) | (x >> jnp.uint32(32 - d))


def _mbits(key, lo):
    k0, k1 = key
    ks0 = jnp.uint32(k0)
    ks1 = jnp.uint32(k1)
    ks2 = jnp.uint32(k0 ^ k1 ^ 0x1BD11BDA)
    x0 = jnp.zeros_like(lo) + ks0
    x1 = lo + ks1
    schedule = ((_ROT_A, ks1, ks2, 1), (_ROT_B, ks2, ks0, 2),
                (_ROT_A, ks0, ks1, 3), (_ROT_B, ks1, ks2, 4),
                (_ROT_A, ks2, ks0, 5))
    for rots, ka, kb, c in schedule:
        for r in rots:
            x0 = x0 + x1
            x1 = _rotl(x1, r)
            x1 = x0 ^ x1
        x0 = x0 + ka
        x1 = x1 + kb + jnp.uint32(c)
    return ((x0 ^ x1) >> jnp.uint32(9)).astype(jnp.int32)


def _gumbel_from_mbits(mbits):
    tiny = jnp.float32(_TINY)
    scale = jnp.float32(1.0) - tiny
    fb = jax.lax.bitcast_convert_type(
        mbits | jnp.int32(0x3F800000), jnp.float32) - jnp.float32(1.0)
    u = jnp.maximum(tiny, fb * scale + tiny)
    return -jnp.log(-jnp.log(u))


def _sample_one(row_u, a, x0, *, n_vocab, n_pad, mask_tok, width, key, chunk):
    n_base = row_u * jnp.uint32(n_vocab)

    best_m = jnp.full((1, width), -1, jnp.int32)
    best_c = jnp.zeros((1, width), jnp.int32)
    s_idx = jax.lax.broadcasted_iota(jnp.int32, (chunk, width), 0)
    s_u = jax.lax.broadcasted_iota(jnp.uint32, (chunk, width), 0)
    for c in range(n_pad // chunk):
        n0 = c * chunk
        n_ids = s_idx + n0
        mb = _mbits(key, n_base + (s_u + jnp.uint32(n0)))
        excl = n_ids == x0
        if n0 <= mask_tok < n0 + chunk:
            excl |= n_ids == mask_tok
        if n0 + chunk > n_vocab:
            excl |= n_ids >= n_vocab
        masked = jnp.where(excl, jnp.int32(-1), mb)
        m_c = jnp.max(masked, axis=0, keepdims=True)
        upd = m_c > best_m
        best_m = jnp.maximum(best_m, m_c)
        best_c = jnp.where(upd, jnp.int32(c), best_c)

    n_ids2 = best_c * chunk + s_idx
    mb2 = _mbits(key, n_base + n_ids2.astype(jnp.uint32))
    excl2 = (n_ids2 == x0) | (n_ids2 == mask_tok) | (n_ids2 >= n_vocab)
    elig = (~excl2) & (mb2 == best_m)
    n_oth = jnp.min(jnp.where(elig, n_ids2, jnp.int32(_BIG)),
                    axis=0, keepdims=True)

    m_x0 = _mbits(key, n_base + x0.astype(jnp.uint32))
    m_mask = _mbits(key, n_base + jnp.uint32(mask_tok))

    g_x0 = _gumbel_from_mbits(m_x0)
    g_mask = _gumbel_from_mbits(m_mask)
    g_oth = _gumbel_from_mbits(best_m)

    eps = jnp.float32(_EPS)
    one = jnp.float32(1.0)
    x0_is_mask = x0 == mask_tok
    p_mask = jnp.where(x0_is_mask, a + (one - a), one - a)
    c_x0 = jnp.where(x0_is_mask, jnp.float32(-jnp.inf),
                     g_x0 + jnp.log(a + eps))
    c_mask = g_mask + jnp.log(p_mask + eps)
    c_oth = g_oth + jnp.log(a * jnp.float32(0.0) + eps)

    best_v, best_i = c_x0, x0
    mask_i = jnp.full_like(best_i, mask_tok)
    take = (c_mask > best_v) | ((c_mask == best_v) & (mask_i < best_i))
    best_v = jnp.where(take, c_mask, best_v)
    best_i = jnp.where(take, mask_i, best_i)
    take = (c_oth > best_v) | ((c_oth == best_v) & (n_oth < best_i))
    return jnp.where(take, n_oth, best_i)


def _sample_body(tok_s_ref, tok_q_ref, a_ref, out_s_ref, out_q_ref, *,
                 width, chunk):
    q = pl.program_id(0)
    a = a_ref[0]
    row_u = (jnp.uint32(q * width)
             + jax.lax.broadcasted_iota(jnp.uint32, (1, width), 1))
    out_s_ref[0] = _sample_one(
        row_u, a, tok_s_ref[0], n_vocab=_STRUC_VOCAB, n_pad=520,
        mask_tok=_STRUC_MASK, width=width, key=_KEY_STRUC, chunk=chunk)
    out_q_ref[0] = _sample_one(
        row_u, a, tok_q_ref[0], n_vocab=_SEQ_VOCAB, n_pad=40,
        mask_tok=_SEQ_MASK, width=width, key=_KEY_SEQ, chunk=chunk)


def _run(structure, sequence, a_rows, width=512, chunk=8, interpret=False):
    b, l = structure.shape
    rows = b * l
    grid = rows // width
    tok_s = structure.reshape(grid, 1, width).astype(jnp.int32)
    tok_q = sequence.reshape(grid, 1, width).astype(jnp.int32)
    a3 = a_rows.reshape(grid, 1, width).astype(jnp.float32)
    body = functools.partial(_sample_body, width=width, chunk=chunk)
    spec = pl.BlockSpec((1, 1, width), lambda q: (q, 0, 0))
    out_s, out_q = pl.pallas_call(
        body,
        grid=(grid,),
        in_specs=[spec, spec, spec],
        out_specs=[spec, spec],
        out_shape=[jax.ShapeDtypeStruct((grid, 1, width), jnp.int32),
                   jax.ShapeDtypeStruct((grid, 1, width), jnp.int32)],
        compiler_params=pltpu.CompilerParams(
            dimension_semantics=("parallel",)),
        interpret=interpret,
    )(tok_s, tok_q, a3)
    return out_s.reshape(b, l), out_q.reshape(b, l)


def kernel(structure, sequence, t, alpha):
    b, l = structure.shape
    a = alpha[t]
    a_rows = jnp.broadcast_to(a[:, None], (b, l))
    noised_structure, noised_seq = _run(structure, sequence, a_rows)
    return (noised_structure, noised_seq, t)


# speedup vs baseline: 3.2431x; 1.1460x over previous
import functools

import numpy as np
import jax
import jax.numpy as jnp
from jax.experimental import pallas as pl
from jax.experimental.pallas import tpu as pltpu

_STRUC_VOCAB = 516
_SEQ_VOCAB = 33
_STRUC_MASK = 2
_SEQ_MASK = 32
_EPS = 1e-6
_TINY = float(np.finfo(np.float32).tiny)
_BIG = 1 << 30

_KEY_STRUC = (1832780943, 270669613)
_KEY_SEQ = (64467757, 2916123636)

_ROT_A = (13, 15, 26, 6)
_ROT_B = (17, 29, 16, 24)


def _rotl(x, d):
    return (x << jnp.uint32(d)) | (x >> jnp.uint32(32 - d))


def _mbits_core(k0, k1, k2, x1):
    schedule = ((_ROT_A, k1, k2, 1), (_ROT_B, k2, k0, 2),
                (_ROT_A, k0, k1, 3), (_ROT_B, k1, k2, 4),
                (_ROT_A, k2, k0, 5))
    x0 = k0 + x1
    x1 = x0 ^ _rotl(x1, _ROT_A[0])
    first = True
    for rots, ka, kb, c in schedule:
        for r in rots:
            if first:
                first = False
                continue
            x0 = x0 + x1
            x1 = _rotl(x1, r)
            x1 = x0 ^ x1
        x0 = x0 + ka
        x1 = x1 + kb + jnp.uint32(c)
    return ((x0 ^ x1) >> jnp.uint32(9)).astype(jnp.int32)


def _key_consts(key):
    k0, k1 = key
    return jnp.uint32(k0), jnp.uint32(k1), jnp.uint32(k0 ^ k1 ^ 0x1BD11BDA)


def _gumbel_from_mbits(mbits):
    tiny = jnp.float32(_TINY)
    scale = jnp.float32(1.0) - tiny
    fb = jax.lax.bitcast_convert_type(
        mbits | jnp.int32(0x3F800000), jnp.float32) - jnp.float32(1.0)
    u = jnp.maximum(tiny, fb * scale + tiny)
    return -jnp.log(-jnp.log(u))


def _scan_others(nb1, x0, *, n_vocab, n_pad, mask_tok, width, key, chunk):
    k0, k1, k2 = _key_consts(key)
    s_idx = jax.lax.broadcasted_iota(jnp.int32, (chunk, width), 0)
    s_u = jax.lax.broadcasted_iota(jnp.uint32, (chunk, width), 0)
    acc = jnp.full((chunk, width), -1, jnp.int32)
    first_c = jnp.zeros((chunk, width), jnp.int32)
    for c in range(n_pad // chunk):
        n0 = c * chunk
        x1 = nb1 + (s_u + jnp.uint32(n0))
        mb = _mbits_core(k0, k1, k2, x1)
        n_ids = s_idx + n0
        excl = n_ids == x0
        if n0 <= mask_tok < n0 + chunk:
            excl |= n_ids == mask_tok
        if n0 + chunk > n_vocab:
            excl |= n_ids >= n_vocab
        masked = jnp.where(excl, jnp.int32(-1), mb)
        upd = masked > acc
        acc = jnp.where(upd, masked, acc)
        first_c = jnp.where(upd, jnp.int32(c), first_c)
    best_m = jnp.max(acc, axis=0, keepdims=True)
    n_cand = first_c * chunk + s_idx
    elig = acc == best_m
    n_oth = jnp.min(jnp.where(elig, n_cand, jnp.int32(_BIG)),
                    axis=0, keepdims=True)
    return best_m, n_oth


def _pick(a, x0, g_x0, g_mask, g_oth, n_oth, mask_tok):
    eps = jnp.float32(_EPS)
    one = jnp.float32(1.0)
    x0_is_mask = x0 == mask_tok
    p_mask = jnp.where(x0_is_mask, a + (one - a), one - a)
    c_x0 = jnp.where(x0_is_mask, jnp.float32(-jnp.inf),
                     g_x0 + jnp.log(a + eps))
    c_mask = g_mask + jnp.log(p_mask + eps)
    c_oth = g_oth + jnp.log(a * jnp.float32(0.0) + eps)

    best_v, best_i = c_x0, x0
    mask_i = jnp.full_like(best_i, mask_tok)
    take = (c_mask > best_v) | ((c_mask == best_v) & (mask_i < best_i))
    best_v = jnp.where(take, c_mask, best_v)
    best_i = jnp.where(take, mask_i, best_i)
    take = (c_oth > best_v) | ((c_oth == best_v) & (n_oth < best_i))
    return jnp.where(take, n_oth, best_i)


def _sample_body(tok_s_ref, tok_q_ref, a_ref, out_s_ref, out_q_ref, *,
                 width, chunk):
    q = pl.program_id(0)
    a = a_ref[0]
    x0_s = tok_s_ref[0]
    x0_q = tok_q_ref[0]
    row_u = (jnp.uint32(q * width)
             + jax.lax.broadcasted_iota(jnp.uint32, (1, width), 1))
    nb1_s = row_u * jnp.uint32(_STRUC_VOCAB) + jnp.uint32(_KEY_STRUC[1])
    nb1_q = row_u * jnp.uint32(_SEQ_VOCAB) + jnp.uint32(_KEY_SEQ[1])

    m_oth_s, n_oth_s = _scan_others(
        nb1_s, x0_s, n_vocab=_STRUC_VOCAB, n_pad=520, mask_tok=_STRUC_MASK,
        width=width, key=_KEY_STRUC, chunk=chunk)
    m_oth_q, n_oth_q = _scan_others(
        nb1_q, x0_q, n_vocab=_SEQ_VOCAB, n_pad=40, mask_tok=_SEQ_MASK,
        width=width, key=_KEY_SEQ, chunk=chunk)

    s8 = jax.lax.broadcasted_iota(jnp.int32, (8, width), 0)
    sk0, sk1, sk2 = _key_consts(_KEY_STRUC)
    qk0, qk1, qk2 = _key_consts(_KEY_SEQ)
    bc = lambda v: jnp.broadcast_to(v, (8, width))
    x1p = jnp.where(
        s8 == 0, bc(nb1_s + x0_s.astype(jnp.uint32)),
        jnp.where(s8 == 1, bc(nb1_s + jnp.uint32(_STRUC_MASK)),
                  jnp.where(s8 == 2, bc(nb1_q + x0_q.astype(jnp.uint32)),
                            bc(nb1_q + jnp.uint32(_SEQ_MASK)))))
    in_s = s8 < 2
    k0p = jnp.where(in_s, sk0, qk0)
    k1p = jnp.where(in_s, sk1, qk1)
    k2p = jnp.where(in_s, sk2, qk2)
    mbp = _mbits_core(k0p, k1p, k2p, x1p)
    gin = jnp.where(s8 == 4, bc(m_oth_s),
                    jnp.where(s8 == 5, bc(m_oth_q), mbp))
    g = _gumbel_from_mbits(gin)

    out_s_ref[0] = _pick(a, x0_s, g[0:1], g[1:2], g[4:5], n_oth_s,
                         _STRUC_MASK)
    out_q_ref[0] = _pick(a, x0_q, g[2:3], g[3:4], g[5:6], n_oth_q,
                         _SEQ_MASK)


def _run(structure, sequence, a_rows, width=1024, chunk=8, interpret=False):
    b, l = structure.shape
    rows = b * l
    grid = rows // width
    tok_s = structure.reshape(grid, 1, width).astype(jnp.int32)
    tok_q = sequence.reshape(grid, 1, width).astype(jnp.int32)
    a3 = a_rows.reshape(grid, 1, width).astype(jnp.float32)
    body = functools.partial(_sample_body, width=width, chunk=chunk)
    spec = pl.BlockSpec((1, 1, width), lambda q: (q, 0, 0))
    out_s, out_q = pl.pallas_call(
        body,
        grid=(grid,),
        in_specs=[spec, spec, spec],
        out_specs=[spec, spec],
        out_shape=[jax.ShapeDtypeStruct((grid, 1, width), jnp.int32),
                   jax.ShapeDtypeStruct((grid, 1, width), jnp.int32)],
        compiler_params=pltpu.CompilerParams(
            dimension_semantics=("parallel",)),
        interpret=interpret,
    )(tok_s, tok_q, a3)
    return out_s.reshape(b, l), out_q.reshape(b, l)


def kernel(structure, sequence, t, alpha):
    b, l = structure.shape
    a = alpha[t]
    a_rows = jnp.broadcast_to(a[:, None], (b, l))
    noised_structure, noised_seq = _run(structure, sequence, a_rows)
    return (noised_structure, noised_seq, t)
